# baseline (speedup 1.0000x reference)
import jax
import jax.numpy as jnp
from jax import lax
from jax.experimental import pallas as pl
from jax.experimental.pallas import tpu as pltpu
from jax.experimental.pallas import tpu_sc as plsc

B = 16384
D = 64
NC = 2
NS = 16
NW = NC * NS
PER_W = B // NW
PASS = 32
NPASS = PER_W // PASS
CHUNK = 128
NCHUNK = PER_W // CHUNK
IDX_ROWS = B // CHUNK


def _sc_body(u2d, i2d, j2d, t0, t1, ti, part_hbm, reg_hbm,
             uidx, pidx, nidx, u0_a, u1_a, p_a, n_a, u0_b, u1_b, p_b, n_b,
             part_v, reg_v, sem_a, sem_b):
    c = lax.axis_index("c")
    s = lax.axis_index("s")
    wid = s * NC + c
    rbase = wid * NCHUNK

    pltpu.sync_copy(u2d.at[pl.ds(rbase, NCHUNK)], uidx)
    pltpu.sync_copy(i2d.at[pl.ds(rbase, NCHUNK)], pidx)
    pltpu.sync_copy(j2d.at[pl.ds(rbase, NCHUNK)], nidx)

    bufs = ((u0_a, u1_a, p_a, n_a, sem_a), (u0_b, u1_b, p_b, n_b, sem_b))

    def issue(p):
        u0_v, u1_v, p_v, n_v, sem = bufs[p % 2]

        @pl.loop(0, PASS // 16)
        def _issue(g):
            t = p * (PASS // 16) + g
            k = t >> 3
            off = (t & 7) * 16
            uvec = uidx[k, pl.ds(off, 16)]
            pvec = pidx[k, pl.ds(off, 16)]
            nvec = nidx[k, pl.ds(off, 16)]
            eb = g * 16
            for j in range(16):
                pltpu.async_copy(t0.at[uvec[j]], u0_v.at[eb + j], sem)
                pltpu.async_copy(t1.at[uvec[j]], u1_v.at[eb + j], sem)
                pltpu.async_copy(ti.at[pvec[j]], p_v.at[eb + j], sem)
                pltpu.async_copy(ti.at[nvec[j]], n_v.at[eb + j], sem)

    reg_acc = jnp.zeros((16,), jnp.float32)
    issue(0)
    for p in range(NPASS):
        if p + 1 < NPASS:
            issue(p + 1)

        u0_v, u1_v, p_v, n_v, sem = bufs[p % 2]
        pltpu.make_async_copy(t0.at[pl.ds(0, PASS)], u0_v, sem).wait()
        pltpu.make_async_copy(t1.at[pl.ds(0, PASS)], u1_v, sem).wait()
        pltpu.make_async_copy(ti.at[pl.ds(0, PASS)], p_v, sem).wait()
        pltpu.make_async_copy(ti.at[pl.ds(0, PASS)], n_v, sem).wait()

        def body(e, reg_acc):
            acc = jnp.zeros((16,), jnp.float32)
            for cc in range(D // 16):
                sl = pl.ds(16 * cc, 16)
                uc = u0_v[e, sl] + u1_v[e, sl]
                pc = p_v[e, sl]
                nc = n_v[e, sl]
                acc = acc + uc * (nc - pc)
                reg_acc = reg_acc + uc * uc
            ee = p * PASS + e
            part_v[ee >> 3, pl.ds((ee & 7) * 16, 16)] = acc
            return reg_acc

        reg_acc = lax.fori_loop(0, PASS, body, reg_acc)

    reg_v[...] = reg_acc
    pltpu.sync_copy(part_v, part_hbm.at[pl.ds(wid * (PER_W // 8), PER_W // 8)])
    pltpu.sync_copy(reg_v, reg_hbm.at[wid])


_sc_kernel = pl.kernel(
    _sc_body,
    out_type=(jax.ShapeDtypeStruct((B // 8, 128), jnp.float32),
              jax.ShapeDtypeStruct((NW, 16), jnp.float32)),
    mesh=plsc.VectorSubcoreMesh(core_axis_name="c", subcore_axis_name="s",
                                num_cores=NC, num_subcores=NS),
    scratch_types=[
        pltpu.VMEM((NCHUNK, CHUNK), jnp.int32),
        pltpu.VMEM((NCHUNK, CHUNK), jnp.int32),
        pltpu.VMEM((NCHUNK, CHUNK), jnp.int32),
        pltpu.VMEM((PASS, D), jnp.float32),
        pltpu.VMEM((PASS, D), jnp.float32),
        pltpu.VMEM((PASS, D), jnp.float32),
        pltpu.VMEM((PASS, D), jnp.float32),
        pltpu.VMEM((PASS, D), jnp.float32),
        pltpu.VMEM((PASS, D), jnp.float32),
        pltpu.VMEM((PASS, D), jnp.float32),
        pltpu.VMEM((PASS, D), jnp.float32),
        pltpu.VMEM((PER_W // 8, 128), jnp.float32),
        pltpu.VMEM((16,), jnp.float32),
        pltpu.SemaphoreType.DMA,
        pltpu.SemaphoreType.DMA,
    ],
    compiler_params=pltpu.CompilerParams(use_tc_tiling_on_sc=True),
)


def _ep_body(part_ref, regp_ref, loss_ref, reg_ref):
    lane = lax.broadcasted_iota(jnp.int32, (128, 8), 0)
    grp = lax.broadcasted_iota(jnp.int32, (128, 8), 1)
    sel = (lane // 16 == grp).astype(jnp.float32)
    x = jnp.dot(part_ref[...], sel,
                preferred_element_type=jnp.float32) * 0.5
    sp = jnp.maximum(x, 0.0) + jnp.log(1.0 + jnp.exp(-jnp.abs(x)))
    loss_ref[...] = jnp.sum(sp, keepdims=True) * (1.0 / B)
    reg_ref[...] = jnp.sum(regp_ref[...], keepdims=True) * (0.125 / B)


_ep_kernel = pl.pallas_call(
    _ep_body,
    out_shape=(jax.ShapeDtypeStruct((1, 1), jnp.float32),
               jax.ShapeDtypeStruct((1, 1), jnp.float32)),
)


def kernel(users, item_i, item_j, user_table0, user_table1, item_table):
    u2d = users.astype(jnp.int32).reshape(IDX_ROWS, CHUNK)
    i2d = item_i.astype(jnp.int32).reshape(IDX_ROWS, CHUNK)
    j2d = item_j.astype(jnp.int32).reshape(IDX_ROWS, CHUNK)
    part_raw, reg_raw = _sc_kernel(u2d, i2d, j2d,
                                   user_table0, user_table1, item_table)
    loss2d, reg2d = _ep_kernel(part_raw, reg_raw)
    return (loss2d[0, 0], reg2d[0, 0])
